# baseline (speedup 1.0000x reference)
import functools

import jax
import jax.numpy as jnp
from jax import lax
from jax.experimental import pallas as pl
from jax.experimental.pallas import tpu as pltpu
from jax.experimental.pallas import tpu_sc as plsc

_DEMB = 768
_W = _DEMB // 2
_NVEC = _DEMB // 32
_NC = 2
_NS = 16
_NW = _NC * _NS
_B = 32768
_BPW = _B // _NW
_C = 32
_NG = 4
_NF = 2
_NCHUNK = _BPW // _C

_mesh = plsc.VectorSubcoreMesh(core_axis_name="c", subcore_axis_name="s")


@functools.partial(
    pl.kernel,
    out_type=jax.ShapeDtypeStruct((_B, _DEMB), jnp.float32),
    mesh=_mesh,
    scratch_types=[
        pltpu.VMEM((_BPW,), jnp.int32),
        pltpu.VMEM((_NG, _C, _W), jnp.int32),
        pltpu.VMEM((_NF, _C, _DEMB), jnp.int32),
        pltpu.SemaphoreType.DMA((_NG,)),
        pltpu.SemaphoreType.DMA((_NF,)),
    ],
)
def _emb_gather(idx_hbm, table_hbm, out_hbm, idx_v, gbuf, fbuf, gsem, ssem):
    wid = lax.axis_index("s") * _NC + lax.axis_index("c")
    base = wid * _BPW
    pltpu.sync_copy(idx_hbm.at[pl.ds(base, _BPW)], idx_v)

    def gather_handle(c):
        b = c % _NG
        return pltpu.make_async_copy(
            table_hbm.at[idx_v.at[pl.ds(c * _C, _C)]], gbuf.at[b], gsem.at[b]
        )

    def store_handle(c):
        b = c % _NF
        return pltpu.make_async_copy(
            fbuf.at[b],
            out_hbm.bitcast(jnp.int32).at[pl.ds(base + c * _C, _C)],
            ssem.at[b],
        )

    for c in range(_NG - 1):
        gather_handle(c).start()
    for c in range(_NCHUNK):
        gather_handle(c).wait()
        g = c + _NG - 1
        if g < _NCHUNK:
            gather_handle(g).start()
        if c - _NF >= 0:
            store_handle(c - _NF).wait()
        gb = gbuf.at[c % _NG]
        fb = fbuf.at[c % _NF]

        def row_body(r, _, gb=gb, fb=fb):
            for j in range(_NVEC):
                v = gb[r, pl.ds(j * 16, 16)]
                fb[r, pl.ds(j * 32, 16)] = v << 16
                fb[r, pl.ds(j * 32 + 16, 16)] = v & jnp.int32(-65536)
            return 0

        lax.fori_loop(0, _C, row_body, 0)
        store_handle(c).start()
    for c in range(_NCHUNK - _NF, _NCHUNK):
        store_handle(c).wait()


def kernel(pos_seq, pos_emb):
    v_rows, d = pos_emb.shape
    idx = pos_seq.reshape(-1).astype(jnp.int32)
    bt = pos_emb.astype(jnp.bfloat16).reshape(v_rows, d // 32, 2, 16)
    bt = bt.transpose(0, 1, 3, 2)
    ti32 = jax.lax.bitcast_convert_type(bt.reshape(v_rows, d // 2, 2), jnp.int32)
    out = _emb_gather(idx, ti32)
    return out.reshape(pos_seq.shape + (d,))


# speedup vs baseline: 1.6929x; 1.6929x over previous
import functools

import jax
import jax.numpy as jnp
from jax import lax
from jax.experimental import pallas as pl
from jax.experimental.pallas import tpu as pltpu
from jax.experimental.pallas import tpu_sc as plsc

_DEMB = 768
_W = _DEMB // 2
_NVEC = _DEMB // 32
_NC = 2
_NS = 16
_NW = _NC * _NS
_B = 32768
_BPW = _B // _NW
_C = 16
_NG = 4
_NF = 2
_NCHUNK = _BPW // _C

_mesh = plsc.VectorSubcoreMesh(core_axis_name="c", subcore_axis_name="s")


@functools.partial(
    pl.kernel,
    out_type=jax.ShapeDtypeStruct((_B, _DEMB), jnp.float32),
    mesh=_mesh,
    scratch_types=[
        pltpu.VMEM((_BPW,), jnp.int32),
        pltpu.VMEM((_NG, _C, _W), jnp.int32),
        pltpu.VMEM((_NF, _C, _DEMB), jnp.int32),
        pltpu.SemaphoreType.DMA((_NG,)),
        pltpu.SemaphoreType.DMA((_NF,)),
    ],
)
def _emb_gather(idx_hbm, table_hbm, out_hbm, idx_v, gbuf, fbuf, gsem, ssem):
    wid = lax.axis_index("s") * _NC + lax.axis_index("c")
    base = wid * _BPW
    pltpu.sync_copy(idx_hbm.at[pl.ds(base, _BPW)], idx_v)

    def gather_handle(c):
        b = c % _NG
        return pltpu.make_async_copy(
            table_hbm.at[idx_v.at[pl.ds(c * _C, _C)]], gbuf.at[b], gsem.at[b]
        )

    def store_handle(c):
        b = c % _NF
        return pltpu.make_async_copy(
            fbuf.at[b],
            out_hbm.bitcast(jnp.int32).at[pl.ds(base + c * _C, _C)],
            ssem.at[b],
        )

    for c in range(_NG - 1):
        gather_handle(c).start()

    def chunk_body(c, carry):
        gather_handle(c).wait()
        g = c + _NG - 1

        @pl.when(g < _NCHUNK)
        def _():
            gather_handle(g).start()

        @pl.when(c >= _NF)
        def _():
            store_handle(c - _NF).wait()

        gb = gbuf.at[c % _NG]
        fb = fbuf.at[c % _NF]

        @plsc.parallel_loop(0, _C)
        def row_body(r, gb=gb, fb=fb):
            for j in range(_NVEC):
                v = gb[r, pl.ds(j * 16, 16)]
                fb[r, pl.ds(j * 32, 16)] = v << 16
                fb[r, pl.ds(j * 32 + 16, 16)] = v & jnp.int32(-65536)

        store_handle(c).start()
        return carry

    lax.fori_loop(0, _NCHUNK, chunk_body, 0)
    for c in range(_NCHUNK - _NF, _NCHUNK):
        store_handle(c).wait()


def kernel(pos_seq, pos_emb):
    v_rows, d = pos_emb.shape
    idx = pos_seq.reshape(-1).astype(jnp.int32)
    b16 = pos_emb.reshape(v_rows, d // 32, 2, 16).astype(jnp.bfloat16)
    u = jax.lax.bitcast_convert_type(b16, jnp.uint16).astype(jnp.uint32)
    word = u[:, :, 0, :] | (u[:, :, 1, :] << 16)
    ti32 = jax.lax.bitcast_convert_type(word.reshape(v_rows, d // 2), jnp.int32)
    out = _emb_gather(idx, ti32)
    return out.reshape(pos_seq.shape + (d,))


# speedup vs baseline: 2.2370x; 1.3214x over previous
import functools

import jax
import jax.numpy as jnp
from jax import lax
from jax.experimental import pallas as pl
from jax.experimental.pallas import tpu as pltpu
from jax.experimental.pallas import tpu_sc as plsc

_DEMB = 768
_W = _DEMB // 2
_NVEC = _W // 16
_NC = 2
_NS = 16
_NW = _NC * _NS
_B = 32768
_BPW = _B // _NW
_C = 16
_NG = 4
_NF = 2
_NCHUNK = _BPW // _C

_mesh = plsc.VectorSubcoreMesh(core_axis_name="c", subcore_axis_name="s")


@functools.partial(
    pl.kernel,
    out_type=jax.ShapeDtypeStruct((_B, _DEMB), jnp.float32),
    mesh=_mesh,
    scratch_types=[
        pltpu.VMEM((_BPW,), jnp.int32),
        pltpu.VMEM((_NG, _C, _W), jnp.int32),
        pltpu.VMEM((_NF, _C, _DEMB), jnp.float32),
        pltpu.SemaphoreType.DMA((_NG,)),
        pltpu.SemaphoreType.DMA((_NF,)),
    ],
)
def _emb_gather(idx_hbm, table_hbm, out_hbm, idx_v, gbuf, fbuf, gsem, ssem):
    wid = lax.axis_index("s") * _NC + lax.axis_index("c")
    base = wid * _BPW
    pltpu.sync_copy(idx_hbm.at[pl.ds(base, _BPW)], idx_v)

    def gather_handle(c):
        b = c % _NG
        return pltpu.make_async_copy(
            table_hbm.at[idx_v.at[pl.ds(c * _C, _C)]], gbuf.at[b], gsem.at[b]
        )

    def store_handle(c):
        b = c % _NF
        return pltpu.make_async_copy(
            fbuf.at[b], out_hbm.at[pl.ds(base + c * _C, _C)], ssem.at[b]
        )

    for c in range(_NG - 1):
        gather_handle(c).start()

    def chunk_body(c, carry):
        gather_handle(c).wait()
        g = c + _NG - 1

        @pl.when(g < _NCHUNK)
        def _():
            gather_handle(g).start()

        @pl.when(c >= _NF)
        def _():
            store_handle(c - _NF).wait()

        gb = gbuf.at[c % _NG]
        fb = fbuf.at[c % _NF]

        @plsc.parallel_loop(0, _C)
        def row_body(r, gb=gb, fb=fb):
            for j in range(_NVEC):
                v = gb[r, pl.ds(j * 16, 16)]
                fb[r, pl.ds(j * 16, 16)] = lax.bitcast_convert_type(
                    v << 16, jnp.float32
                )
                fb[r, pl.ds(_W + j * 16, 16)] = lax.bitcast_convert_type(
                    v & jnp.int32(-65536), jnp.float32
                )

        store_handle(c).start()
        return carry

    lax.fori_loop(0, _NCHUNK, chunk_body, 0)
    for c in range(_NCHUNK - _NF, _NCHUNK):
        store_handle(c).wait()


def kernel(pos_seq, pos_emb):
    v_rows, d = pos_emb.shape
    idx = pos_seq.reshape(-1).astype(jnp.int32)
    u = jax.lax.bitcast_convert_type(
        pos_emb.astype(jnp.bfloat16), jnp.uint16
    ).astype(jnp.uint32)
    word = u[:, : d // 2] | (u[:, d // 2 :] << 16)
    ti32 = jax.lax.bitcast_convert_type(word, jnp.int32)
    out = _emb_gather(idx, ti32)
    return out.reshape(pos_seq.shape + (d,))


# speedup vs baseline: 2.2878x; 1.0227x over previous
import functools

import jax
import jax.numpy as jnp
from jax import lax
from jax.experimental import pallas as pl
from jax.experimental.pallas import tpu as pltpu
from jax.experimental.pallas import tpu_sc as plsc

_DEMB = 768
_NC = 2
_NS = 16
_NW = _NC * _NS
_B = 32768
_BPW = _B // _NW
_C = 16
_NB = 8
_NCHUNK = _BPW // _C

_mesh = plsc.VectorSubcoreMesh(core_axis_name="c", subcore_axis_name="s")


@functools.partial(
    pl.kernel,
    out_type=jax.ShapeDtypeStruct((_B, _DEMB), jnp.float32),
    mesh=_mesh,
    scratch_types=[
        pltpu.VMEM((_BPW,), jnp.int32),
        pltpu.VMEM((_NB, _C, _DEMB), jnp.float32),
        pltpu.SemaphoreType.DMA((_NB,)),
        pltpu.SemaphoreType.DMA((_NB,)),
    ],
)
def _emb_gather(idx_hbm, table_hbm, out_hbm, idx_v, buf, gsem, ssem):
    wid = lax.axis_index("s") * _NC + lax.axis_index("c")
    base = wid * _BPW
    pltpu.sync_copy(idx_hbm.at[pl.ds(base, _BPW)], idx_v)

    def gather_handle(c):
        b = c % _NB
        return pltpu.make_async_copy(
            table_hbm.at[idx_v.at[pl.ds(c * _C, _C)]], buf.at[b], gsem.at[b]
        )

    def store_handle(c):
        b = c % _NB
        return pltpu.make_async_copy(
            buf.at[b], out_hbm.at[pl.ds(base + c * _C, _C)], ssem.at[b]
        )

    for c in range(_NB - 1):
        gather_handle(c).start()

    def chunk_body(c, carry):
        gather_handle(c).wait()
        store_handle(c).start()
        g = c + _NB - 1

        @pl.when(jnp.logical_and(c >= 1, g < _NCHUNK))
        def _():
            store_handle(g - _NB).wait()

        @pl.when(g < _NCHUNK)
        def _():
            gather_handle(g).start()

        return carry

    lax.fori_loop(0, _NCHUNK, chunk_body, 0)
    for c in range(_NCHUNK - _NB, _NCHUNK):
        store_handle(c).wait()


def kernel(pos_seq, pos_emb):
    d = pos_emb.shape[-1]
    idx = pos_seq.reshape(-1).astype(jnp.int32)
    out = _emb_gather(idx, pos_emb)
    return out.reshape(pos_seq.shape + (d,))
